# Initial kernel scaffold; baseline (speedup 1.0000x reference)
import jax
import jax.numpy as jnp
from jax.experimental import pallas as pl


def kernel(feat_topo, sup_topo, feat_gnd, sup_gnd, train_flag, W):
    raise NotImplementedError("write your pallas kernel here")


# speedup vs baseline: 1.0901x; 1.0901x over previous
import functools

import jax
import jax.numpy as jnp
from jax.experimental import pallas as pl
from jax.experimental.pallas import tpu as pltpu

N = 8192
D = 128
BM = 256


def _bf16x3_dot(a, b):
    a_hi = a.astype(jnp.bfloat16)
    a_lo = (a - a_hi.astype(jnp.float32)).astype(jnp.bfloat16)
    b_hi = b.astype(jnp.bfloat16)
    b_lo = (b - b_hi.astype(jnp.float32)).astype(jnp.bfloat16)
    dims = (((1,), (0,)), ((), ()))
    f32 = jnp.float32
    return (
        jax.lax.dot_general(a_hi, b_hi, dims, preferred_element_type=f32)
        + jax.lax.dot_general(a_lo, b_hi, dims, preferred_element_type=f32)
        + jax.lax.dot_general(a_hi, b_lo, dims, preferred_element_type=f32)
    )


def _bf16_dot(a, b):
    return jax.lax.dot_general(
        a.astype(jnp.bfloat16), b.astype(jnp.bfloat16),
        (((1,), (0,)), ((), ())), preferred_element_type=jnp.float32)


def _gnn_block(sup_ref, feat_ref, w_ref, out_ref):
    agg = _bf16_dot(sup_ref[...], feat_ref[...])
    h = _bf16_dot(agg, w_ref[...])
    t = jnp.tanh(h)
    nrm = jnp.sqrt(jnp.sum(t * t, axis=1, keepdims=True))
    out_ref[...] = t / jnp.maximum(nrm, 1e-12)


@functools.partial(jax.jit, static_argnames=())
def _gnn_pipeline(sup, feat, w):
    return pl.pallas_call(
        _gnn_block,
        grid=(N // BM,),
        in_specs=[
            pl.BlockSpec((BM, N), lambda i: (i, 0)),
            pl.BlockSpec((N, D), lambda i: (0, 0)),
            pl.BlockSpec((D, D), lambda i: (0, 0)),
        ],
        out_specs=pl.BlockSpec((BM, D), lambda i: (i, 0)),
        out_shape=jax.ShapeDtypeStruct((N, D), jnp.float32),
        compiler_params=pltpu.CompilerParams(
            dimension_semantics=("parallel",),
        ),
    )(sup, feat, w)


def kernel(feat_topo, sup_topo, feat_gnd, sup_gnd, train_flag, W):
    out_topo = _gnn_pipeline(sup_topo, feat_topo, W)
    out_gnd = _gnn_pipeline(sup_gnd, feat_gnd, W)
    out_gnd = jnp.where(train_flag != 0, out_gnd, jnp.zeros_like(out_gnd))
    return (out_topo, out_gnd)


# speedup vs baseline: 1.0982x; 1.0075x over previous
import functools

import jax
import jax.numpy as jnp
from jax.experimental import pallas as pl
from jax.experimental.pallas import tpu as pltpu

N = 8192
D = 128
BM = 512


def _bf16x3_dot(a, b):
    a_hi = a.astype(jnp.bfloat16)
    a_lo = (a - a_hi.astype(jnp.float32)).astype(jnp.bfloat16)
    b_hi = b.astype(jnp.bfloat16)
    b_lo = (b - b_hi.astype(jnp.float32)).astype(jnp.bfloat16)
    dims = (((1,), (0,)), ((), ()))
    f32 = jnp.float32
    return (
        jax.lax.dot_general(a_hi, b_hi, dims, preferred_element_type=f32)
        + jax.lax.dot_general(a_lo, b_hi, dims, preferred_element_type=f32)
        + jax.lax.dot_general(a_hi, b_lo, dims, preferred_element_type=f32)
    )


def _bf16_dot(a, b):
    return jax.lax.dot_general(
        a.astype(jnp.bfloat16), b.astype(jnp.bfloat16),
        (((1,), (0,)), ((), ())), preferred_element_type=jnp.float32)


def _gnn_block(sup_ref, feat_ref, w_ref, out_ref):
    agg = _bf16_dot(sup_ref[...], feat_ref[...])
    h = _bf16_dot(agg, w_ref[...])
    t = jnp.tanh(h)
    nrm = jnp.sqrt(jnp.sum(t * t, axis=1, keepdims=True))
    out_ref[...] = t / jnp.maximum(nrm, 1e-12)


@functools.partial(jax.jit, static_argnames=())
def _gnn_pipeline(sup, feat, w):
    return pl.pallas_call(
        _gnn_block,
        grid=(N // BM,),
        in_specs=[
            pl.BlockSpec((BM, N), lambda i: (i, 0)),
            pl.BlockSpec((N, D), lambda i: (0, 0)),
            pl.BlockSpec((D, D), lambda i: (0, 0)),
        ],
        out_specs=pl.BlockSpec((BM, D), lambda i: (i, 0)),
        out_shape=jax.ShapeDtypeStruct((N, D), jnp.float32),
        compiler_params=pltpu.CompilerParams(
            dimension_semantics=("parallel",),
        ),
    )(sup, feat, w)


def kernel(feat_topo, sup_topo, feat_gnd, sup_gnd, train_flag, W):
    out_topo = _gnn_pipeline(sup_topo, feat_topo, W)
    out_gnd = _gnn_pipeline(sup_gnd, feat_gnd, W)
    out_gnd = jnp.where(train_flag != 0, out_gnd, jnp.zeros_like(out_gnd))
    return (out_topo, out_gnd)


# speedup vs baseline: 1.1361x; 1.0345x over previous
import jax
import jax.numpy as jnp
from jax.experimental import pallas as pl
from jax.experimental.pallas import tpu as pltpu

N = 8192
D = 128
BM = 256


def _bf16_dot(a, b):
    return jax.lax.dot_general(
        a.astype(jnp.bfloat16), b.astype(jnp.bfloat16),
        (((1,), (0,)), ((), ())), preferred_element_type=jnp.float32)


def _pipeline_block(sup, feat, w):
    agg = _bf16_dot(sup, feat)
    t = jnp.tanh(_bf16_dot(agg, w))
    nrm = jnp.sqrt(jnp.sum(t * t, axis=1, keepdims=True))
    return t / jnp.maximum(nrm, 1e-12)


def _gnn_block(sup_t_ref, sup_g_ref, feat_t_ref, feat_g_ref, w_ref,
               out_t_ref, out_g_ref):
    w = w_ref[...]
    out_t_ref[...] = _pipeline_block(sup_t_ref[...], feat_t_ref[...], w)
    out_g_ref[...] = _pipeline_block(sup_g_ref[...], feat_g_ref[...], w)


def kernel(feat_topo, sup_topo, feat_gnd, sup_gnd, train_flag, W):
    sup_spec = pl.BlockSpec((BM, N), lambda i: (i, 0))
    feat_spec = pl.BlockSpec((N, D), lambda i: (0, 0))
    out_spec = pl.BlockSpec((BM, D), lambda i: (i, 0))
    out_topo, out_gnd = pl.pallas_call(
        _gnn_block,
        grid=(N // BM,),
        in_specs=[sup_spec, sup_spec, feat_spec, feat_spec,
                  pl.BlockSpec((D, D), lambda i: (0, 0))],
        out_specs=[out_spec, out_spec],
        out_shape=[jax.ShapeDtypeStruct((N, D), jnp.float32),
                   jax.ShapeDtypeStruct((N, D), jnp.float32)],
        compiler_params=pltpu.CompilerParams(
            dimension_semantics=("parallel",),
        ),
    )(sup_topo, sup_gnd, feat_topo, feat_gnd, W)
    out_gnd = jnp.where(train_flag != 0, out_gnd, jnp.zeros_like(out_gnd))
    return (out_topo, out_gnd)
